# Initial kernel scaffold; baseline (speedup 1.0000x reference)
import jax
import jax.numpy as jnp
from jax.experimental import pallas as pl


def kernel(neighbors, table):
    raise NotImplementedError("write your pallas kernel here")


# speedup vs baseline: 1.5751x; 1.5751x over previous
import functools

import jax
import jax.numpy as jnp
from jax import lax
from jax.experimental import pallas as pl
from jax.experimental.pallas import tpu as pltpu
from jax.experimental.pallas import tpu_sc as plsc

NC = 2
NS = 16
L = 16
NW = NC * NS
S = 32
D = 128
NODES_PER_W = 320
CHUNK_NODES = 4
ROWS_PER_CHUNK = CHUNK_NODES * S
NCHUNK = NODES_PER_W // CHUNK_NODES
N_PAD = NW * NODES_PER_W

_mesh = plsc.VectorSubcoreMesh(
    core_axis_name="c", subcore_axis_name="s", num_cores=NC)


@functools.partial(
    pl.kernel,
    out_type=jax.ShapeDtypeStruct((N_PAD, D), jnp.float32),
    mesh=_mesh,
    scratch_types=[
        pltpu.VMEM((NCHUNK, ROWS_PER_CHUNK), jnp.int32),
        pltpu.VMEM((ROWS_PER_CHUNK, D), jnp.float32),
        pltpu.VMEM((ROWS_PER_CHUNK, D), jnp.float32),
        pltpu.VMEM((NODES_PER_W, D), jnp.float32),
        pltpu.SemaphoreType.DMA,
        pltpu.SemaphoreType.DMA,
    ],
)
def _agg_kernel(idx_hbm, table_hbm, out_hbm,
                idx_v, buf_a, buf_b, out_v, sem_a, sem_b):
    wid = lax.axis_index("s") * NC + lax.axis_index("c")

    pltpu.sync_copy(idx_hbm.at[wid], idx_v)

    bufs = (buf_a, buf_b)
    sems = (sem_a, sem_b)

    def start(chunk, b):
        pltpu.make_async_copy(
            table_hbm.at[idx_v.at[chunk]], bufs[b], sems[b]).start()

    def wait(b):
        pltpu.make_async_copy(
            table_hbm.at[pl.ds(0, ROWS_PER_CHUNK)], bufs[b], sems[b]).wait()

    def compute(b, cur):
        rows = bufs[b]
        for i in range(CHUNK_NODES):
            base = i * S
            accs = [rows[base, pl.ds(c * L, L)] for c in range(D // L)]
            for j in range(1, S):
                for c in range(D // L):
                    accs[c] = accs[c] + rows[base + j, pl.ds(c * L, L)]
            node = cur * CHUNK_NODES + i
            for c in range(D // L):
                out_v[node, pl.ds(c * L, L)] = accs[c] * (1.0 / S)

    start(0, 0)

    def body(g, carry):
        c0 = 2 * g
        c1 = c0 + 1
        start(c1, 1)
        wait(0)
        compute(0, c0)

        @pl.when(c1 + 1 < NCHUNK)
        def _():
            start(c1 + 1, 0)

        wait(1)
        compute(1, c1)
        return carry

    lax.fori_loop(0, NCHUNK // 2, body, 0)

    pltpu.sync_copy(out_v, out_hbm.at[pl.ds(wid * NODES_PER_W, NODES_PER_W)])


def kernel(neighbors, table):
    n, _ = neighbors.shape
    idx = neighbors.astype(jnp.int32)
    idx = jnp.pad(idx, ((0, N_PAD - n), (0, 0)))
    idx3 = idx.reshape(NW, NCHUNK, ROWS_PER_CHUNK)
    out = _agg_kernel(idx3, table)
    return out[:n]


# speedup vs baseline: 5.5062x; 3.4957x over previous
import functools

import jax
import jax.numpy as jnp
from jax import lax
from jax.experimental import pallas as pl
from jax.experimental.pallas import tpu as pltpu
from jax.experimental.pallas import tpu_sc as plsc

NC = 2
NS = 16
L = 16
NW = NC * NS
S = 32
D = 128
NODES_PER_W = 320
CHUNK_NODES = 4
ROWS_PER_CHUNK = CHUNK_NODES * S
NCHUNK = NODES_PER_W // CHUNK_NODES
GROUP_CHUNKS = 4
NGROUP = NCHUNK // GROUP_CHUNKS
FLUSH_NODES = 2 * CHUNK_NODES
N_PAD = NW * NODES_PER_W
N_TABLE = N_PAD
ROWS_PER_STAGER = N_TABLE // NS

_mesh = plsc.VectorSubcoreMesh(
    core_axis_name="c", subcore_axis_name="s", num_cores=NC)


@functools.partial(
    pl.kernel,
    out_type=jax.ShapeDtypeStruct((N_PAD, D), jnp.float32),
    mesh=_mesh,
    scratch_types=[
        pltpu.VMEM_SHARED((N_TABLE, D), jnp.float32),
        pltpu.VMEM((NCHUNK, ROWS_PER_CHUNK), jnp.int32),
        pltpu.VMEM((ROWS_PER_CHUNK, D), jnp.float32),
        pltpu.VMEM((ROWS_PER_CHUNK, D), jnp.float32),
        pltpu.VMEM((FLUSH_NODES, D), jnp.float32),
        pltpu.VMEM((FLUSH_NODES, D), jnp.float32),
        pltpu.SemaphoreType.DMA,
        pltpu.SemaphoreType.DMA,
        pltpu.SemaphoreType.DMA,
        pltpu.SemaphoreType.DMA,
    ],
)
def _agg_kernel(idx_hbm, table_hbm, out_hbm,
                table_sp, idx_v, buf_a, buf_b, fl_a, fl_b,
                sem_a, sem_b, fsem_a, fsem_b):
    sid = lax.axis_index("s")
    wid = sid * NC + lax.axis_index("c")
    out_base = wid * NODES_PER_W

    pltpu.sync_copy(table_hbm.at[pl.ds(sid * ROWS_PER_STAGER, ROWS_PER_STAGER)],
                    table_sp.at[pl.ds(sid * ROWS_PER_STAGER, ROWS_PER_STAGER)])
    pltpu.sync_copy(idx_hbm.at[wid], idx_v)
    plsc.subcore_barrier()

    bufs = (buf_a, buf_b)
    sems = (sem_a, sem_b)
    fls = (fl_a, fl_b)
    fsems = (fsem_a, fsem_b)

    def start(chunk, b):
        pltpu.make_async_copy(
            table_sp.at[idx_v.at[chunk]], bufs[b], sems[b]).start()

    def wait(b):
        pltpu.make_async_copy(
            table_hbm.at[pl.ds(0, ROWS_PER_CHUNK)], bufs[b], sems[b]).wait()

    def compute(b, fbuf, row0):
        rows = bufs[b]
        for i in range(CHUNK_NODES):
            base = i * S
            accs = tuple(rows[base, pl.ds(c * L, L)] for c in range(D // L))

            def acc_body(j, accs):
                return tuple(a + rows[base + j, pl.ds(c * L, L)]
                             for c, a in enumerate(accs))

            accs = lax.fori_loop(1, S, acc_body, accs, unroll=8)
            for c in range(D // L):
                fbuf[row0 + i, pl.ds(c * L, L)] = accs[c] * (1.0 / S)

    start(0, 0)

    def body(g, carry):
        k0 = GROUP_CHUNKS * g
        for f in range(2):
            fbuf, fsem = fls[f], fsems[f]
            @pl.when(g >= 1)
            def _():
                pltpu.make_async_copy(
                    fbuf, out_hbm.at[pl.ds(0, FLUSH_NODES)], fsem).wait()
            for p in range(2):
                k = k0 + 2 * f + p
                b = (2 * f + p) % 2
                @pl.when(k + 1 < NCHUNK)
                def _():
                    start(k + 1, 1 - b)
                wait(b)
                compute(b, fbuf, p * CHUNK_NODES)
            pltpu.make_async_copy(
                fbuf,
                out_hbm.at[pl.ds(out_base + (2 * g + f) * FLUSH_NODES,
                                 FLUSH_NODES)],
                fsem).start()
        return carry

    lax.fori_loop(0, NGROUP, body, 0)
    for f in range(2):
        pltpu.make_async_copy(
            fls[f], out_hbm.at[pl.ds(0, FLUSH_NODES)], fsems[f]).wait()


def kernel(neighbors, table):
    n, _ = neighbors.shape
    idx = neighbors.astype(jnp.int32)
    idx = jnp.pad(idx, ((0, N_PAD - n), (0, 0)))
    idx3 = idx.reshape(NW, NCHUNK, ROWS_PER_CHUNK)
    table_p = jnp.pad(table, ((0, N_TABLE - table.shape[0]), (0, 0)))
    out = _agg_kernel(idx3, table_p)
    return out[:n]


# speedup vs baseline: 8.3187x; 1.5108x over previous
import functools

import jax
import jax.numpy as jnp
from jax import lax
from jax.experimental import pallas as pl
from jax.experimental.pallas import tpu as pltpu
from jax.experimental.pallas import tpu_sc as plsc

NC = 2
NS = 16
L = 16
NW = NC * NS
S = 32
D = 128
NODES_PER_W = 320
NBLK = 3
BLK_ROWS = (128, 128, 64)
BLK_PAD = 128
N_PAD = NW * NODES_PER_W
N_TABLE = N_PAD
ROWS_PER_STAGER = N_TABLE // NS
WINDOW = 8

_mesh = plsc.VectorSubcoreMesh(
    core_axis_name="c", subcore_axis_name="s", num_cores=NC)


@functools.partial(
    pl.kernel,
    out_type=jax.ShapeDtypeStruct((N_PAD, D), jnp.float32),
    mesh=_mesh,
    scratch_types=[
        pltpu.VMEM_SHARED((N_TABLE, D), jnp.float32),
        pltpu.VMEM((NBLK, S, BLK_PAD), jnp.int32),
        pltpu.VMEM((BLK_PAD, D), jnp.float32),
        pltpu.VMEM((BLK_PAD, D), jnp.float32),
        pltpu.SemaphoreType.DMA,
        pltpu.SemaphoreType.DMA,
        pltpu.SemaphoreType.DMA,
    ],
)
def _agg_kernel(idx_hbm, table_hbm, out_hbm,
                table_sp, idx_v, acc_a, acc_b, gsem, fsem_a, fsem_b):
    sid = lax.axis_index("s")
    wid = sid * NC + lax.axis_index("c")
    out_base = wid * NODES_PER_W

    pltpu.sync_copy(table_hbm.at[pl.ds(sid * ROWS_PER_STAGER, ROWS_PER_STAGER)],
                    table_sp.at[pl.ds(sid * ROWS_PER_STAGER, ROWS_PER_STAGER)])
    pltpu.sync_copy(idx_hbm.at[wid], idx_v)
    plsc.subcore_barrier()

    accs = (acc_a, acc_b)
    fsems = (fsem_a, fsem_b)

    for nb in range(NBLK):
        b = nb & 1
        n_rows = BLK_ROWS[nb]
        acc = accs[b]
        dst = acc.at[pl.ds(0, n_rows)]

        def start(j, add):
            pltpu.make_async_copy(
                table_sp.at[idx_v.at[nb, j, pl.ds(0, n_rows)]], dst,
                gsem).start(add=add)

        def wait_one():
            pltpu.make_async_copy(
                table_hbm.at[pl.ds(0, n_rows)], dst, gsem).wait()

        if nb >= 2:
            pltpu.make_async_copy(
                accs[0].at[pl.ds(0, BLK_ROWS[0])],
                out_hbm.at[pl.ds(0, BLK_ROWS[0])], fsems[0]).wait()

        start(0, False)
        wait_one()

        def pass_body(j, carry):
            start(j, True)

            @pl.when(j > WINDOW)
            def _():
                wait_one()

            return carry

        lax.fori_loop(1, S, pass_body, 0)
        for _ in range(WINDOW):
            wait_one()

        def scale_body(r, carry):
            for c in range(D // L):
                acc[r, pl.ds(c * L, L)] = acc[r, pl.ds(c * L, L)] * (1.0 / S)
            return carry

        lax.fori_loop(0, n_rows, scale_body, 0, unroll=4)

        pltpu.make_async_copy(
            dst, out_hbm.at[pl.ds(out_base + nb * BLK_PAD, n_rows)],
            fsems[b]).start()

    pltpu.make_async_copy(
        accs[1].at[pl.ds(0, BLK_ROWS[1])],
        out_hbm.at[pl.ds(0, BLK_ROWS[1])], fsems[1]).wait()
    pltpu.make_async_copy(
        accs[0].at[pl.ds(0, BLK_ROWS[2])],
        out_hbm.at[pl.ds(0, BLK_ROWS[2])], fsems[0]).wait()


def kernel(neighbors, table):
    n, _ = neighbors.shape
    idx = neighbors.astype(jnp.int32)
    idx = jnp.pad(idx, ((0, N_PAD - n), (0, 0)))
    idx = idx.reshape(NW, NODES_PER_W, S)
    idx = jnp.pad(idx, ((0, 0), (0, NBLK * BLK_PAD - NODES_PER_W), (0, 0)))
    idx4 = idx.reshape(NW, NBLK, BLK_PAD, S).transpose(0, 1, 3, 2)
    table_p = jnp.pad(table, ((0, N_TABLE - table.shape[0]), (0, 0)))
    out = _agg_kernel(idx4, table_p)
    return out[:n]


# speedup vs baseline: 8.6278x; 1.0372x over previous
import functools

import jax
import jax.numpy as jnp
from jax import lax
from jax.experimental import pallas as pl
from jax.experimental.pallas import tpu as pltpu
from jax.experimental.pallas import tpu_sc as plsc

NC = 2
NS = 16
L = 16
NW = NC * NS
S = 32
D = 128
NODES_PER_W = 320
NBLK = 3
BLK_ROWS = (128, 128, 64)
BLK_PAD = 128
N_PAD = NW * NODES_PER_W
N_TABLE = 10000
ROWS_PER_STAGER = 632
WINDOW = 8

_mesh = plsc.VectorSubcoreMesh(
    core_axis_name="c", subcore_axis_name="s", num_cores=NC)


@functools.partial(
    pl.kernel,
    out_type=jax.ShapeDtypeStruct((N_PAD, D), jnp.float32),
    mesh=_mesh,
    scratch_types=[
        pltpu.VMEM_SHARED((N_TABLE, D), jnp.float32),
        pltpu.VMEM((NBLK, S, BLK_PAD), jnp.int32),
        pltpu.VMEM((BLK_PAD, D), jnp.float32),
        pltpu.VMEM((BLK_PAD, D), jnp.float32),
        pltpu.SemaphoreType.DMA,
        pltpu.SemaphoreType.DMA,
        pltpu.SemaphoreType.DMA,
        pltpu.SemaphoreType.DMA,
    ],
)
def _agg_kernel(idx_hbm, table_hbm, out_hbm,
                table_sp, idx_v, acc_a, acc_b, gsem_a, gsem_b,
                fsem_a, fsem_b):
    sid = lax.axis_index("s")
    wid = sid * NC + lax.axis_index("c")
    out_base = wid * NODES_PER_W

    stage_base = pl.multiple_of(
        jnp.minimum(sid * ROWS_PER_STAGER, N_TABLE - ROWS_PER_STAGER), 8)
    pltpu.sync_copy(table_hbm.at[pl.ds(stage_base, ROWS_PER_STAGER)],
                    table_sp.at[pl.ds(stage_base, ROWS_PER_STAGER)])
    pltpu.sync_copy(idx_hbm.at[wid], idx_v)
    plsc.subcore_barrier()

    accs = (acc_a, acc_b)
    gsems = (gsem_a, gsem_b)
    fsems = (fsem_a, fsem_b)
    zeros = jnp.zeros((L,), jnp.float32)

    def gather_wait(b, n_rows):
        pltpu.make_async_copy(
            table_hbm.at[pl.ds(0, n_rows)],
            accs[b].at[pl.ds(0, n_rows)], gsems[b]).wait()

    def launch_block(nb):
        b = nb & 1
        n_rows = BLK_ROWS[nb]
        acc = accs[b]
        dst = acc.at[pl.ds(0, n_rows)]

        def zero_body(r, carry):
            for c in range(D // L):
                acc[r, pl.ds(c * L, L)] = zeros
            return carry

        lax.fori_loop(0, n_rows, zero_body, 0, unroll=4)

        def pass_body(j, carry):
            pltpu.make_async_copy(
                table_sp.at[idx_v.at[nb, j, pl.ds(0, n_rows)]], dst,
                gsems[b]).start(add=True)

            @pl.when(j >= WINDOW)
            def _():
                gather_wait(b, n_rows)

            return carry

        lax.fori_loop(0, S, pass_body, 0)

    def finish_block(nb):
        b = nb & 1
        n_rows = BLK_ROWS[nb]
        acc = accs[b]
        for _ in range(WINDOW):
            gather_wait(b, n_rows)

        def scale_body(r, carry):
            for c in range(D // L):
                acc[r, pl.ds(c * L, L)] = acc[r, pl.ds(c * L, L)] * (1.0 / S)
            return carry

        lax.fori_loop(0, n_rows, scale_body, 0, unroll=4)

        pltpu.make_async_copy(
            acc.at[pl.ds(0, n_rows)],
            out_hbm.at[pl.ds(out_base + nb * BLK_PAD, n_rows)],
            fsems[b]).start()

    def flush_wait(b, n_rows):
        pltpu.make_async_copy(
            accs[b].at[pl.ds(0, n_rows)],
            out_hbm.at[pl.ds(0, n_rows)], fsems[b]).wait()

    launch_block(0)
    launch_block(1)
    finish_block(0)
    flush_wait(0, BLK_ROWS[0])
    launch_block(2)
    finish_block(1)
    finish_block(2)
    flush_wait(1, BLK_ROWS[1])
    flush_wait(0, BLK_ROWS[2])


def kernel(neighbors, table):
    n, _ = neighbors.shape
    idx = neighbors.astype(jnp.int32)
    idx = jnp.pad(idx, ((0, N_PAD - n), (0, 0)))
    idx = idx.reshape(NW, NODES_PER_W, S)
    idx = jnp.pad(idx, ((0, 0), (0, NBLK * BLK_PAD - NODES_PER_W), (0, 0)))
    idx4 = idx.reshape(NW, NBLK, BLK_PAD, S).transpose(0, 1, 3, 2)
    out = _agg_kernel(idx4, table)
    return out[:n]


# speedup vs baseline: 9.0394x; 1.0477x over previous
import functools

import jax
import jax.numpy as jnp
from jax import lax
from jax.experimental import pallas as pl
from jax.experimental.pallas import tpu as pltpu
from jax.experimental.pallas import tpu_sc as plsc

NC = 2
NS = 16
L = 16
NW = NC * NS
S = 32
D = 128
NODES_PER_W = 320
NBLK = 3
BLK_ROWS = (128, 128, 64)
BLK_PAD = 128
N_PAD = NW * NODES_PER_W
N_TABLE = 10000
ROWS_PER_STAGER = 632
WINDOW = 8

_mesh = plsc.VectorSubcoreMesh(
    core_axis_name="c", subcore_axis_name="s", num_cores=NC)


N_OUT = 10000


@functools.partial(
    pl.kernel,
    out_type=(jax.ShapeDtypeStruct((N_OUT, D), jnp.float32),
              jax.ShapeDtypeStruct((BLK_PAD, D), jnp.float32)),
    mesh=_mesh,
    scratch_types=[
        pltpu.VMEM_SHARED((N_TABLE, D), jnp.float32),
        pltpu.VMEM((NBLK, S, BLK_PAD), jnp.int32),
        pltpu.VMEM((BLK_PAD, D), jnp.float32),
        pltpu.VMEM((BLK_PAD, D), jnp.float32),
        pltpu.SemaphoreType.DMA,
        pltpu.SemaphoreType.DMA,
        pltpu.SemaphoreType.DMA,
        pltpu.SemaphoreType.DMA,
    ],
)
def _agg_kernel(idx_hbm, table_hbm, out_hbm, spill_hbm,
                table_sp, idx_v, acc_a, acc_b, gsem_a, gsem_b,
                fsem_a, fsem_b):
    sid = lax.axis_index("s")
    wid = sid * NC + lax.axis_index("c")
    out_base = wid * NODES_PER_W

    stage_base = pl.multiple_of(
        jnp.minimum(sid * ROWS_PER_STAGER, N_TABLE - ROWS_PER_STAGER), 8)
    pltpu.sync_copy(table_hbm.at[pl.ds(stage_base, ROWS_PER_STAGER)],
                    table_sp.at[pl.ds(stage_base, ROWS_PER_STAGER)])
    pltpu.sync_copy(idx_hbm.at[wid], idx_v)
    plsc.subcore_barrier()

    accs = (acc_a, acc_b)
    gsems = (gsem_a, gsem_b)
    fsems = (fsem_a, fsem_b)
    zeros = jnp.zeros((L,), jnp.float32)

    def gather_wait(b, n_rows):
        pltpu.make_async_copy(
            table_hbm.at[pl.ds(0, n_rows)],
            accs[b].at[pl.ds(0, n_rows)], gsems[b]).wait()

    def launch_block(nb):
        b = nb & 1
        n_rows = BLK_ROWS[nb]
        acc = accs[b]
        dst = acc.at[pl.ds(0, n_rows)]

        def zero_body(r, carry):
            for c in range(D // L):
                acc[r, pl.ds(c * L, L)] = zeros
            return carry

        lax.fori_loop(0, n_rows, zero_body, 0, unroll=4)

        def pass_body(j, carry):
            pltpu.make_async_copy(
                table_sp.at[idx_v.at[nb, j, pl.ds(0, n_rows)]], dst,
                gsems[b]).start(add=True)

            @pl.when(j >= WINDOW)
            def _():
                gather_wait(b, n_rows)

            return carry

        lax.fori_loop(0, S, pass_body, 0)

    def finish_block(nb):
        b = nb & 1
        n_rows = BLK_ROWS[nb]
        acc = accs[b]
        for _ in range(WINDOW):
            gather_wait(b, n_rows)

        def scale_body(r, carry):
            for c in range(D // L):
                acc[r, pl.ds(c * L, L)] = acc[r, pl.ds(c * L, L)] * (1.0 / S)
            return carry

        lax.fori_loop(0, n_rows, scale_body, 0, unroll=4)

        @pl.when(wid < NW - 1)
        def _():
            pltpu.make_async_copy(
                acc.at[pl.ds(0, n_rows)],
                out_hbm.at[pl.ds(out_base + nb * BLK_PAD, n_rows)],
                fsems[b]).start()

        @pl.when(wid == NW - 1)
        def _():
            if nb == 0:
                pltpu.make_async_copy(
                    acc.at[pl.ds(0, N_OUT - (NW - 1) * NODES_PER_W)],
                    out_hbm.at[pl.ds((NW - 1) * NODES_PER_W,
                                     N_OUT - (NW - 1) * NODES_PER_W)],
                    fsems[b]).start()
            else:
                pltpu.make_async_copy(
                    acc.at[pl.ds(0, n_rows)],
                    spill_hbm.at[pl.ds(0, n_rows)], fsems[b]).start()

    def flush_wait(b, n_rows):
        pltpu.make_async_copy(
            accs[b].at[pl.ds(0, n_rows)],
            out_hbm.at[pl.ds(0, n_rows)], fsems[b]).wait()

    launch_block(0)
    launch_block(1)
    finish_block(0)
    @pl.when(wid < NW - 1)
    def _():
        flush_wait(0, BLK_ROWS[0])

    @pl.when(wid == NW - 1)
    def _():
        flush_wait(0, N_OUT - (NW - 1) * NODES_PER_W)

    launch_block(2)
    finish_block(1)
    finish_block(2)
    flush_wait(1, BLK_ROWS[1])
    flush_wait(0, BLK_ROWS[2])


def kernel(neighbors, table):
    n, _ = neighbors.shape
    idx = neighbors.astype(jnp.int32)
    idx = jnp.pad(idx, ((0, N_PAD - n), (0, 0)))
    idx = idx.reshape(NW, NODES_PER_W, S)
    idx = jnp.pad(idx, ((0, 0), (0, NBLK * BLK_PAD - NODES_PER_W), (0, 0)))
    idx4 = idx.reshape(NW, NBLK, BLK_PAD, S).transpose(0, 1, 3, 2)
    out, _ = _agg_kernel(idx4, table)
    return out


# speedup vs baseline: 9.2168x; 1.0196x over previous
import functools

import jax
import jax.numpy as jnp
from jax import lax
from jax.experimental import pallas as pl
from jax.experimental.pallas import tpu as pltpu
from jax.experimental.pallas import tpu_sc as plsc

NC = 2
NS = 16
L = 16
NW = NC * NS
S = 32
D = 128
NODES_PER_W = 320
NBLK = 3
BLK_ROWS = (128, 128, 64)
BLK_PAD = 128
N_PAD = NW * NODES_PER_W
N_TABLE = 10000
ROWS_PER_STAGER = 632
WINDOW = 8

_mesh = plsc.VectorSubcoreMesh(
    core_axis_name="c", subcore_axis_name="s", num_cores=NC)


N_OUT = 10000


@functools.partial(
    pl.kernel,
    out_type=(jax.ShapeDtypeStruct((N_OUT, D), jnp.float32),
              jax.ShapeDtypeStruct((BLK_PAD, D), jnp.float32)),
    mesh=_mesh,
    scratch_types=[
        pltpu.VMEM_SHARED((N_TABLE, D), jnp.float32),
        pltpu.VMEM((NBLK, S, BLK_PAD), jnp.int32),
        pltpu.VMEM((BLK_PAD, D), jnp.float32),
        pltpu.VMEM((BLK_PAD, D), jnp.float32),
        pltpu.SemaphoreType.DMA,
        pltpu.SemaphoreType.DMA,
        pltpu.SemaphoreType.DMA,
        pltpu.SemaphoreType.DMA,
        pltpu.SemaphoreType.DMA,
    ],
)
def _agg_kernel(idx_hbm, table_hbm, out_hbm, spill_hbm,
                table_sp, idx_v, acc_a, acc_b, gsem_a, gsem_b,
                fsem_a, fsem_b, tsem):
    sid = lax.axis_index("s")
    wid = sid * NC + lax.axis_index("c")
    out_base = wid * NODES_PER_W

    accs = (acc_a, acc_b)
    gsems = (gsem_a, gsem_b)
    fsems = (fsem_a, fsem_b)
    zeros = jnp.zeros((L,), jnp.float32)

    def zero_acc(nb):
        acc = accs[nb & 1]

        def zero_body(r, carry):
            for c in range(D // L):
                acc[r, pl.ds(c * L, L)] = zeros
            return carry

        lax.fori_loop(0, BLK_ROWS[nb], zero_body, 0, unroll=4)

    stage_base = pl.multiple_of(
        jnp.minimum(sid * ROWS_PER_STAGER, N_TABLE - ROWS_PER_STAGER), 8)
    stage = pltpu.make_async_copy(
        table_hbm.at[pl.ds(stage_base, ROWS_PER_STAGER)],
        table_sp.at[pl.ds(stage_base, ROWS_PER_STAGER)], tsem)
    stage.start()
    pltpu.sync_copy(idx_hbm.at[wid], idx_v)
    zero_acc(0)
    zero_acc(1)
    stage.wait()
    plsc.subcore_barrier()

    def gather_wait(b, n_rows):
        pltpu.make_async_copy(
            table_hbm.at[pl.ds(0, n_rows)],
            accs[b].at[pl.ds(0, n_rows)], gsems[b]).wait()

    def launch_block(nb, zero=False):
        b = nb & 1
        n_rows = BLK_ROWS[nb]
        acc = accs[b]
        dst = acc.at[pl.ds(0, n_rows)]
        if zero:
            zero_acc(nb)

        def pass_body(j, carry):
            pltpu.make_async_copy(
                table_sp.at[idx_v.at[nb, j, pl.ds(0, n_rows)]], dst,
                gsems[b]).start(add=True)

            @pl.when(j >= WINDOW)
            def _():
                gather_wait(b, n_rows)

            return carry

        lax.fori_loop(0, S, pass_body, 0)

    def finish_block(nb):
        b = nb & 1
        n_rows = BLK_ROWS[nb]
        acc = accs[b]
        for _ in range(WINDOW):
            gather_wait(b, n_rows)

        def scale_body(r, carry):
            for c in range(D // L):
                acc[r, pl.ds(c * L, L)] = acc[r, pl.ds(c * L, L)] * (1.0 / S)
            return carry

        lax.fori_loop(0, n_rows, scale_body, 0, unroll=4)

        @pl.when(wid < NW - 1)
        def _():
            pltpu.make_async_copy(
                acc.at[pl.ds(0, n_rows)],
                out_hbm.at[pl.ds(out_base + nb * BLK_PAD, n_rows)],
                fsems[b]).start()

        @pl.when(wid == NW - 1)
        def _():
            if nb == 0:
                pltpu.make_async_copy(
                    acc.at[pl.ds(0, N_OUT - (NW - 1) * NODES_PER_W)],
                    out_hbm.at[pl.ds((NW - 1) * NODES_PER_W,
                                     N_OUT - (NW - 1) * NODES_PER_W)],
                    fsems[b]).start()
            else:
                pltpu.make_async_copy(
                    acc.at[pl.ds(0, n_rows)],
                    spill_hbm.at[pl.ds(0, n_rows)], fsems[b]).start()

    def flush_wait(b, n_rows):
        pltpu.make_async_copy(
            accs[b].at[pl.ds(0, n_rows)],
            out_hbm.at[pl.ds(0, n_rows)], fsems[b]).wait()

    launch_block(0)
    launch_block(1)
    finish_block(0)
    @pl.when(wid < NW - 1)
    def _():
        flush_wait(0, BLK_ROWS[0])

    @pl.when(wid == NW - 1)
    def _():
        flush_wait(0, N_OUT - (NW - 1) * NODES_PER_W)

    launch_block(2, zero=True)
    finish_block(1)
    finish_block(2)
    flush_wait(1, BLK_ROWS[1])
    flush_wait(0, BLK_ROWS[2])


def kernel(neighbors, table):
    n, _ = neighbors.shape
    idx = neighbors.astype(jnp.int32)
    idx = jnp.pad(idx, ((0, N_PAD - n), (0, 0)))
    idx = idx.reshape(NW, NODES_PER_W, S)
    idx = jnp.pad(idx, ((0, 0), (0, NBLK * BLK_PAD - NODES_PER_W), (0, 0)))
    idx4 = idx.reshape(NW, NBLK, BLK_PAD, S).transpose(0, 1, 3, 2)
    out, _ = _agg_kernel(idx4, table)
    return out
